# baseline (speedup 1.0000x reference)
import functools

import jax
import jax.numpy as jnp
from jax import lax
from jax.experimental import pallas as pl
from jax.experimental.pallas import tpu as pltpu
from jax.experimental.pallas import tpu_sc as plsc

D = 128
DW = D // 2
LANES = 16

_info = plsc.get_sparse_core_info()
NC, NS = _info.num_cores, _info.num_subcores
NW = NC * NS

CHUNK = 80
GROUPS = CHUNK // LANES
PSTRIDE = CHUNK + 5
CW = 2 * CHUNK


def _make_score_kernel(num_edges, num_nodes):
    assert num_edges % NW == 0
    per_w = num_edges // NW
    assert per_w % (2 * CHUNK) == 0
    n_chunks = per_w // CHUNK
    rows_per_sub = 2 * num_nodes // NS
    assert 2 * num_nodes % NS == 0 and rows_per_sub % 125 == 0
    assert num_nodes == (NS // 2) * rows_per_sub

    mesh = plsc.VectorSubcoreMesh(core_axis_name="c", subcore_axis_name="s")

    @functools.partial(
        pl.kernel,
        mesh=mesh,
        compiler_params=pltpu.CompilerParams(
            needs_layout_passes=False, use_tc_tiling_on_sc=False),
        out_type=jax.ShapeDtypeStruct((num_edges,), jnp.float32),
        scratch_types=[
            pltpu.VMEM((2, CW), jnp.int32),
            pltpu.VMEM((2, CW, DW), jnp.int32),
            pltpu.VMEM((2, CHUNK), jnp.float32),
            pltpu.VMEM((LANES * PSTRIDE,), jnp.float32),
            pltpu.VMEM((DW,), jnp.int32),
            pltpu.VMEM_SHARED((2 * num_nodes, DW), jnp.int32),
            pltpu.SemaphoreType.DMA((2,)),
            pltpu.SemaphoreType.DMA((2,)),
            pltpu.SemaphoreType.DMA((2,)),
        ],
    )
    def score_kernel(cidx_hbm, xw_hbm, r_hbm, out_hbm,
                     cidx, rows, scores, pmat, r_v,
                     xw_sh, sem_rows, sem_idx, sem_out):
        sid = lax.axis_index("s")
        wid = sid * NC + lax.axis_index("c")
        base = wid * per_w

        pltpu.sync_copy(r_hbm, r_v)
        r_bf = [plsc.bitcast(r_v[pl.ds(LANES * j, LANES)], jnp.bfloat16)
                for j in range(DW // LANES)]

        row0 = sid * rows_per_sub

        @pl.when(sid < NS // 2)
        def _():
            pltpu.sync_copy(xw_hbm.at[pl.ds(row0, rows_per_sub)],
                            xw_sh.at[pl.ds(row0, rows_per_sub)])

        @pl.when(sid >= NS // 2)
        def _():
            scale_rows = 125
            vtmp = rows.at[0, pl.ds(0, scale_rows)]
            for t in range(rows_per_sub // scale_rows):
                sl = pl.ds(row0 + t * scale_rows, scale_rows)
                pltpu.sync_copy(xw_hbm.at[sl], vtmp)

                @plsc.parallel_loop(0, scale_rows, 1, unroll=4)
                def scale_body(row):
                    for j in range(DW // LANES):
                        w = plsc.bitcast(
                            vtmp[row, pl.ds(LANES * j, LANES)], jnp.bfloat16)
                        vtmp[row, pl.ds(LANES * j, LANES)] = plsc.bitcast(
                            w * r_bf[j], jnp.int32)

                pltpu.sync_copy(vtmp, xw_sh.at[sl])

        plsc.subcore_barrier()

        lane = lax.broadcasted_iota(jnp.int32, (LANES,), 0)
        col_base = lane * PSTRIDE

        def start_idx(c, b):
            off = (wid * n_chunks + c) * CW
            pltpu.async_copy(cidx_hbm.at[pl.ds(off, CW)],
                             cidx.at[b], sem_idx.at[b])

        def wait_idx(b):
            pltpu.make_async_copy(cidx_hbm.at[pl.ds(0, CW)],
                                  cidx.at[b], sem_idx.at[b]).wait()

        NSPLIT = 4
        SLEN = CW // NSPLIT

        def start_rows(b):
            for h in range(NSPLIT):
                s = pl.ds(h * SLEN, SLEN)
                pltpu.async_copy(xw_sh.at[cidx.at[b, s]],
                                 rows.at[b, s], sem_rows.at[b])

        def wait_rows(b):
            for h in range(NSPLIT):
                s = pl.ds(h * SLEN, SLEN)
                pltpu.make_async_copy(xw_sh.at[cidx.at[b, s]],
                                      rows.at[b, s], sem_rows.at[b]).wait()

        def compute(c, b):
            @plsc.parallel_loop(0, CHUNK, 1, unroll=4)
            def edge_body(e):
                parts = []
                for j in range(DW // LANES):
                    u = plsc.bitcast(
                        rows[b, e, pl.ds(LANES * j, LANES)], jnp.bfloat16)
                    v = plsc.bitcast(
                        rows[b, CHUNK + e, pl.ds(LANES * j, LANES)],
                        jnp.bfloat16)
                    wa, wb = plsc.unpack(
                        u * v, format=plsc.PackFormat.INTERLEAVED)
                    parts.append(wa)
                    parts.append(wb)
                while len(parts) > 1:
                    parts = [a + bb for a, bb in
                             zip(parts[::2], parts[1::2])]
                plsc.store_scatter(pmat, [col_base + e], parts[0])

            @plsc.parallel_loop(0, GROUPS, 1)
            def group_body(gb):
                gbase = gb * LANES + lane
                accs = [plsc.load_gather(pmat, [gbase + l * PSTRIDE])
                        for l in range(4)]
                for l in range(4, LANES):
                    accs[l % 4] = accs[l % 4] + plsc.load_gather(
                        pmat, [gbase + l * PSTRIDE])
                acc = (accs[0] + accs[1]) + (accs[2] + accs[3])
                scores[b, pl.ds(gb * LANES, LANES)] = acc

            pltpu.async_copy(scores.at[b],
                             out_hbm.at[pl.ds(base + c * CHUNK, CHUNK)],
                             sem_out.at[b])

        def wait_out(b):
            pltpu.make_async_copy(scores.at[b],
                                  out_hbm.at[pl.ds(0, CHUNK)],
                                  sem_out.at[b]).wait()

        start_idx(0, 0)
        start_idx(1, 1)
        wait_idx(0)
        start_rows(0)

        def loop_body(i, carry):
            for b in (0, 1):
                c = 2 * i + b
                wait_rows(b)

                @pl.when(c + 2 < n_chunks)
                def _():
                    start_idx(c + 2, b)

                @pl.when(c + 1 < n_chunks)
                def _():
                    wait_idx(1 - b)
                    start_rows(1 - b)

                @pl.when(c >= 2)
                def _():
                    wait_out(b)

                compute(c, b)
            return carry

        lax.fori_loop(0, n_chunks // 2, loop_body, 0)
        wait_out(0)
        wait_out(1)

    return score_kernel


def kernel(positive_edges, negative_edges, g, x_u, x_v, r):
    e = positive_edges.shape[0]
    n = x_u.shape[0]
    srcs = jnp.concatenate([positive_edges[:, 0], negative_edges[:, 0]])
    dsts = jnp.concatenate([positive_edges[:, 1], negative_edges[:, 1]])
    cidx = jnp.concatenate(
        [srcs.reshape(-1, CHUNK), dsts.reshape(-1, CHUNK) + n],
        axis=1).reshape(-1)

    def to_words(t):
        t16 = t.astype(jnp.bfloat16)
        return lax.bitcast_convert_type(
            t16.reshape(*t16.shape[:-1], DW, 2), jnp.int32)

    xw = jnp.concatenate([to_words(x_u), to_words(x_v)], axis=0)
    scores = _make_score_kernel(2 * e, n)(cidx, xw, to_words(r))
    return (scores[:e], scores[e:])


# speedup vs baseline: 1.1733x; 1.1733x over previous
import functools

import jax
import jax.numpy as jnp
from jax import lax
from jax.experimental import pallas as pl
from jax.experimental.pallas import tpu as pltpu
from jax.experimental.pallas import tpu_sc as plsc

D = 128
DW = D // 2
LANES = 16

_info = plsc.get_sparse_core_info()
NC, NS = _info.num_cores, _info.num_subcores
NW = NC * NS

CHUNK = 80
GROUPS = CHUNK // LANES
PSTRIDE = CHUNK + 5


def _make_score_kernel(num_edges, num_nodes):
    assert num_edges % NW == 0
    assert num_nodes % NS == 0
    rows_per_sub = num_nodes // NS
    per_w = num_edges // NW
    assert per_w % (2 * CHUNK) == 0
    n_chunks = per_w // CHUNK

    mesh = plsc.VectorSubcoreMesh(core_axis_name="c", subcore_axis_name="s")

    @functools.partial(
        pl.kernel,
        mesh=mesh,
        compiler_params=pltpu.CompilerParams(
            needs_layout_passes=False, use_tc_tiling_on_sc=False),
        out_type=jax.ShapeDtypeStruct((num_edges,), jnp.float32),
        scratch_types=[
            pltpu.VMEM((2, CHUNK), jnp.int32),
            pltpu.VMEM((2, CHUNK), jnp.int32),
            pltpu.VMEM((2, CHUNK, DW), jnp.int32),
            pltpu.VMEM((2, CHUNK, DW), jnp.int32),
            pltpu.VMEM((2, CHUNK), jnp.float32),
            pltpu.VMEM((LANES * PSTRIDE,), jnp.float32),
            pltpu.VMEM((DW,), jnp.int32),
            pltpu.VMEM((125, DW), jnp.int32),
            pltpu.VMEM_SHARED((num_nodes, DW), jnp.int32),
            pltpu.VMEM_SHARED((num_nodes, DW), jnp.int32),
            pltpu.SemaphoreType.DMA((2,)),
            pltpu.SemaphoreType.DMA((2,)),
            pltpu.SemaphoreType.DMA((2,)),
        ],
    )
    def score_kernel(srcs_hbm, dsts_hbm, xu_hbm, xv_hbm, r_hbm, out_hbm,
                     src_idx, dst_idx, u_rows, v_rows, scores, pmat, r_v,
                     vtmp, xu_sh, xv_sh, sem_rows, sem_idx, sem_out):
        sid = lax.axis_index("s")
        wid = sid * NC + lax.axis_index("c")
        base = wid * per_w

        row0 = sid * rows_per_sub
        pltpu.sync_copy(xu_hbm.at[pl.ds(row0, rows_per_sub)],
                        xu_sh.at[pl.ds(row0, rows_per_sub)])

        pltpu.sync_copy(r_hbm, r_v)
        r_bf = [plsc.bitcast(r_v[pl.ds(LANES * j, LANES)], jnp.bfloat16)
                for j in range(DW // LANES)]

        scale_rows = 125
        assert rows_per_sub % scale_rows == 0
        for t in range(rows_per_sub // scale_rows):
            r0 = row0 + t * scale_rows
            sl = pl.ds(r0, scale_rows)
            pltpu.sync_copy(xv_hbm.at[sl], vtmp)

            @plsc.parallel_loop(0, scale_rows, 1, unroll=4)
            def scale_body(row):
                for j in range(DW // LANES):
                    w = plsc.bitcast(vtmp[row, pl.ds(LANES * j, LANES)],
                                     jnp.bfloat16)
                    vtmp[row, pl.ds(LANES * j, LANES)] = plsc.bitcast(
                        w * r_bf[j], jnp.int32)

            pltpu.sync_copy(vtmp, xv_sh.at[sl])
        plsc.subcore_barrier()

        lane = lax.broadcasted_iota(jnp.int32, (LANES,), 0)
        col_base = lane * PSTRIDE

        def start_idx(c, b):
            off = base + c * CHUNK
            pltpu.async_copy(srcs_hbm.at[pl.ds(off, CHUNK)],
                             src_idx.at[b], sem_idx.at[b])
            pltpu.async_copy(dsts_hbm.at[pl.ds(off, CHUNK)],
                             dst_idx.at[b], sem_idx.at[b])

        def wait_idx(b):
            pltpu.make_async_copy(srcs_hbm.at[pl.ds(0, CHUNK)],
                                  src_idx.at[b], sem_idx.at[b]).wait()
            pltpu.make_async_copy(srcs_hbm.at[pl.ds(0, CHUNK)],
                                  dst_idx.at[b], sem_idx.at[b]).wait()

        HALF = CHUNK // 2

        def start_rows(b):
            for h in (0, 1):
                s = pl.ds(h * HALF, HALF)
                pltpu.async_copy(xu_sh.at[src_idx.at[b, s]],
                                 u_rows.at[b, s], sem_rows.at[b])
                pltpu.async_copy(xv_sh.at[dst_idx.at[b, s]],
                                 v_rows.at[b, s], sem_rows.at[b])

        def wait_rows(b):
            for h in (0, 1):
                s = pl.ds(h * HALF, HALF)
                pltpu.make_async_copy(xu_sh.at[src_idx.at[b, s]],
                                      u_rows.at[b, s], sem_rows.at[b]).wait()
                pltpu.make_async_copy(xv_sh.at[dst_idx.at[b, s]],
                                      v_rows.at[b, s], sem_rows.at[b]).wait()

        def compute(c, b):
            @plsc.parallel_loop(0, CHUNK, 1, unroll=4)
            def edge_body(e):
                parts = []
                for j in range(DW // LANES):
                    u = plsc.bitcast(
                        u_rows[b, e, pl.ds(LANES * j, LANES)],
                        jnp.bfloat16)
                    v = plsc.bitcast(
                        v_rows[b, e, pl.ds(LANES * j, LANES)],
                        jnp.bfloat16)
                    wa, wb = plsc.unpack(
                        u * v, format=plsc.PackFormat.INTERLEAVED)
                    parts.append(wa)
                    parts.append(wb)
                while len(parts) > 1:
                    parts = [a + bb for a, bb in
                             zip(parts[::2], parts[1::2])]
                plsc.store_scatter(pmat, [col_base + e], parts[0])

            @plsc.parallel_loop(0, GROUPS, 1)
            def group_body(gb):
                gbase = gb * LANES + lane
                accs = [plsc.load_gather(pmat, [gbase + l * PSTRIDE])
                        for l in range(4)]
                for l in range(4, LANES):
                    accs[l % 4] = accs[l % 4] + plsc.load_gather(
                        pmat, [gbase + l * PSTRIDE])
                acc = (accs[0] + accs[1]) + (accs[2] + accs[3])
                scores[b, pl.ds(gb * LANES, LANES)] = acc
            pltpu.async_copy(scores.at[b],
                             out_hbm.at[pl.ds(base + c * CHUNK, CHUNK)],
                             sem_out.at[b])

        def wait_out(c, b):
            pltpu.make_async_copy(scores.at[b],
                                  out_hbm.at[pl.ds(0, CHUNK)],
                                  sem_out.at[b]).wait()

        start_idx(0, 0)
        start_idx(1, 1)
        wait_idx(0)
        start_rows(0)

        def loop_body(i, carry):
            for b in (0, 1):
                c = 2 * i + b
                wait_rows(b)

                @pl.when(c + 2 < n_chunks)
                def _():
                    start_idx(c + 2, b)

                @pl.when(c + 1 < n_chunks)
                def _():
                    wait_idx(1 - b)
                    start_rows(1 - b)

                @pl.when(c >= 2)
                def _():
                    wait_out(c, b)

                compute(c, b)
            return carry

        lax.fori_loop(0, n_chunks // 2, loop_body, 0)
        wait_out(n_chunks - 2, 0)
        wait_out(n_chunks - 1, 1)

    return score_kernel


def kernel(positive_edges, negative_edges, g, x_u, x_v, r):
    e = positive_edges.shape[0]
    n = x_u.shape[0]
    srcs = jnp.concatenate([positive_edges[:, 0], negative_edges[:, 0]])
    dsts = jnp.concatenate([positive_edges[:, 1], negative_edges[:, 1]])

    def to_words(t):
        t16 = t.astype(jnp.bfloat16)
        return lax.bitcast_convert_type(
            t16.reshape(*t16.shape[:-1], DW, 2), jnp.int32)

    scores = _make_score_kernel(2 * e, n)(
        srcs, dsts, to_words(x_u), to_words(x_v), to_words(r))
    return (scores[:e], scores[e:])
